# Initial kernel scaffold; baseline (speedup 1.0000x reference)
import jax
import jax.numpy as jnp
from jax.experimental import pallas as pl


def kernel(latent_z, emb_weight):
    raise NotImplementedError("write your pallas kernel here")


# speedup vs baseline: 5.8110x; 5.8110x over previous
import functools

import jax
import jax.numpy as jnp
from jax.experimental import pallas as pl
from jax.experimental.pallas import tpu as pltpu

_N_EMB = 8192
_DIM = 32
_N_TOK = 8192
_R = 256
_N_BLK = _N_TOK // _R
_BETA = 0.25


def _vq_body(z_ref, emb_ref, z2_ref, e2_ref, enc_ref, idx_ref, q_ref,
             loss_ref, perp_ref, counts_ref, loss_acc_ref):
    i = pl.program_id(0)
    z = z_ref[...]
    emb = emb_ref[...]

    z2 = z2_ref[...]
    e2 = e2_ref[...]
    m = jax.lax.dot_general(
        z, emb, (((1,), (1,)), ((), ())),
        preferred_element_type=jnp.float32)
    d = (z2 + e2) - 2.0 * m

    rowmin = jnp.min(d, axis=1, keepdims=True)
    col = jax.lax.broadcasted_iota(jnp.int32, (_R, _N_EMB), 1)
    idx = jnp.min(jnp.where(d == rowmin, col, _N_EMB),
                  axis=1, keepdims=True)
    onehot = (col == idx).astype(jnp.float32)

    enc_ref[...] = onehot
    idx_ref[...] = idx

    q = jax.lax.dot_general(
        onehot, emb, (((1,), (0,)), ((), ())),
        preferred_element_type=jnp.float32)
    q_ref[...] = z + (q - z)

    @pl.when(i == 0)
    def _init():
        counts_ref[...] = jnp.zeros_like(counts_ref)
        loss_acc_ref[...] = jnp.zeros_like(loss_acc_ref)

    counts_ref[...] += jnp.sum(onehot, axis=0, keepdims=True)
    diff = q - z
    loss_acc_ref[...] += jnp.sum(diff * diff).reshape(1, 1)

    @pl.when(i == _N_BLK - 1)
    def _fini():
        m1 = loss_acc_ref[...] * (1.0 / (_N_TOK * _DIM))
        loss_ref[...] = m1 + _BETA * m1
        p = counts_ref[...] * (1.0 / _N_TOK)
        ent = -jnp.sum(p * jnp.log(p + 1e-10))
        perp_ref[...] = jnp.exp(ent).reshape(1, 1)


@functools.partial(jax.jit, static_argnames=("interpret",))
def _vq_call(flat_z, emb_weight, z2, e2, interpret=False):
    out_shapes = (
        jax.ShapeDtypeStruct((_N_TOK, _N_EMB), jnp.float32),
        jax.ShapeDtypeStruct((_N_TOK, 1), jnp.int32),
        jax.ShapeDtypeStruct((_N_TOK, _DIM), jnp.float32),
        jax.ShapeDtypeStruct((1, 1), jnp.float32),
        jax.ShapeDtypeStruct((1, 1), jnp.float32),
    )
    grid = (_N_BLK,)
    return pl.pallas_call(
        _vq_body,
        grid=grid,
        in_specs=[
            pl.BlockSpec((_R, _DIM), lambda i: (i, 0)),
            pl.BlockSpec((_N_EMB, _DIM), lambda i: (0, 0)),
            pl.BlockSpec((_R, 1), lambda i: (i, 0)),
            pl.BlockSpec((1, _N_EMB), lambda i: (0, 0)),
        ],
        out_specs=(
            pl.BlockSpec((_R, _N_EMB), lambda i: (i, 0)),
            pl.BlockSpec((_R, 1), lambda i: (i, 0)),
            pl.BlockSpec((_R, _DIM), lambda i: (i, 0)),
            pl.BlockSpec((1, 1), lambda i: (0, 0)),
            pl.BlockSpec((1, 1), lambda i: (0, 0)),
        ),
        out_shape=out_shapes,
        scratch_shapes=[
            pltpu.VMEM((1, _N_EMB), jnp.float32),
            pltpu.VMEM((1, 1), jnp.float32),
        ],
        interpret=interpret,
    )(flat_z, emb_weight, z2, e2)


def kernel(latent_z, emb_weight):
    z = jnp.transpose(latent_z, (0, 2, 3, 1))
    flat_z = z.reshape(_N_TOK, _DIM)
    z2 = jnp.sum(flat_z ** 2, axis=1, keepdims=True)
    e2 = jnp.sum(emb_weight ** 2, axis=1)[None, :]
    enc, idx, q_flat, loss, perp = _vq_call(flat_z, emb_weight, z2, e2)
    quantized_z = jnp.transpose(q_flat.reshape(z.shape), (0, 3, 1, 2))
    return (loss.reshape(()), quantized_z, perp.reshape(()),
            enc, idx)
